# baseline (speedup 1.0000x reference)
import functools

import jax
import jax.numpy as jnp
from jax import lax
from jax.experimental import pallas as pl
from jax.experimental.pallas import tpu as pltpu
from jax.experimental.pallas import tpu_sc as plsc

NC = 2
NS = 16
NW = NC * NS
NBUF = 2


@functools.partial(jax.jit, static_argnames=("chunk",))
def _emb_lookup(idx_flat, table, chunk):
    n = idx_flat.shape[0]
    d = table.shape[1]
    bpw = n // NW
    nchunk = bpw // chunk
    assert nchunk % NBUF == 0
    mesh = plsc.VectorSubcoreMesh(core_axis_name="c", subcore_axis_name="s")

    @functools.partial(
        pl.kernel,
        mesh=mesh,
        out_type=jax.ShapeDtypeStruct((n, d), jnp.float32),
        scratch_types=(
            [pltpu.VMEM((chunk,), jnp.int32) for _ in range(NBUF)]
            + [pltpu.VMEM((chunk, d), jnp.float32) for _ in range(NBUF)]
            + [pltpu.SemaphoreType.DMA for _ in range(2 * NBUF)]
        ),
        compiler_params=pltpu.CompilerParams(use_tc_tiling_on_sc=False),
    )
    def emb(idx_hbm, table_hbm, out_hbm, *scratch):
        idx_v = scratch[:NBUF]
        rows_v = scratch[NBUF:2 * NBUF]
        gsem = scratch[2 * NBUF:3 * NBUF]
        wsem = scratch[3 * NBUF:]
        wid = lax.axis_index("s") * NC + lax.axis_index("c")
        base = wid * bpw

        def chunk_off(g):
            return pl.multiple_of(base + g * chunk, 8)

        def start_gather(g, b):
            off = chunk_off(g)
            pltpu.sync_copy(idx_hbm.at[pl.ds(off, chunk)], idx_v[b])
            pltpu.async_copy(table_hbm.at[idx_v[b]], rows_v[b], gsem[b])

        for b in range(NBUF):
            start_gather(b, b)

        def group(j, _):
            for b in range(NBUF):
                g = j * NBUF + b
                pltpu.make_async_copy(
                    table_hbm.at[idx_v[b]], rows_v[b], gsem[b]
                ).wait()
                wcopy = pltpu.async_copy(
                    rows_v[b], out_hbm.at[pl.ds(chunk_off(g), chunk)], wsem[b]
                )
                wcopy.wait()

                @pl.when(g + NBUF < nchunk)
                def _():
                    start_gather(g + NBUF, b)
            return 0

        lax.fori_loop(0, nchunk // NBUF, group, 0)

    return emb(idx_flat, table)


def kernel(input_ids, table):
    b, h = input_ids.shape
    idx_flat = input_ids.reshape(b * h).astype(jnp.int32)
    out = _emb_lookup(idx_flat, table, chunk=1600)
    return out.reshape(b, h, table.shape[1])


# speedup vs baseline: 1.6224x; 1.6224x over previous
import functools

import jax
import jax.numpy as jnp
from jax import lax
from jax.experimental import pallas as pl
from jax.experimental.pallas import tpu as pltpu
from jax.experimental.pallas import tpu_sc as plsc

NC = 2
NS = 16
NW = NC * NS
NBUF = 2


@functools.partial(jax.jit, static_argnames=("bblk", "bsz", "hist"))
def _emb_lookup(idx_flat, table, bblk, bsz, hist):
    d = table.shape[1]
    chunk = bblk * hist
    b_per_w = bsz // NW
    nchunk = b_per_w // bblk
    assert nchunk % NBUF == 0
    mesh = plsc.VectorSubcoreMesh(core_axis_name="c", subcore_axis_name="s")

    @functools.partial(
        pl.kernel,
        mesh=mesh,
        out_type=jax.ShapeDtypeStruct((bsz, hist, d), jnp.float32),
        scratch_types=(
            [pltpu.VMEM((chunk,), jnp.int32) for _ in range(NBUF)]
            + [pltpu.VMEM((chunk, d), jnp.float32) for _ in range(NBUF)]
            + [pltpu.SemaphoreType.DMA for _ in range(2 * NBUF)]
        ),
        compiler_params=pltpu.CompilerParams(use_tc_tiling_on_sc=False),
    )
    def emb(idx_hbm, table_hbm, out_hbm, *scratch):
        idx_v = scratch[:NBUF]
        rows_v = scratch[NBUF:2 * NBUF]
        gsem = scratch[2 * NBUF:3 * NBUF]
        wsem = scratch[3 * NBUF:]
        wid = lax.axis_index("s") * NC + lax.axis_index("c")
        bbase = wid * b_per_w
        fbase = bbase * hist

        def start_gather(g, b):
            off = pl.multiple_of(fbase + g * chunk, 8)
            pltpu.sync_copy(idx_hbm.at[pl.ds(off, chunk)], idx_v[b])
            pltpu.async_copy(table_hbm.at[idx_v[b]], rows_v[b], gsem[b])

        for b in range(NBUF):
            start_gather(b, b)

        def group(j, _):
            for b in range(NBUF):
                g = j * NBUF + b
                pltpu.make_async_copy(
                    table_hbm.at[idx_v[b]], rows_v[b], gsem[b]
                ).wait()
                b0 = bbase + g * bblk
                for j2 in range(bblk):
                    pltpu.async_copy(
                        rows_v[b].at[pl.ds(j2 * hist, hist)],
                        out_hbm.at[b0 + j2],
                        wsem[b],
                    )
                for j2 in range(bblk):
                    pltpu.make_async_copy(
                        rows_v[b].at[pl.ds(j2 * hist, hist)],
                        out_hbm.at[b0 + j2],
                        wsem[b],
                    ).wait()

                @pl.when(g + NBUF < nchunk)
                def _():
                    start_gather(g + NBUF, b)
            return 0

        lax.fori_loop(0, nchunk // NBUF, group, 0)

    return emb(idx_flat, table)


def kernel(input_ids, table):
    b, h = input_ids.shape
    return _emb_lookup(input_ids.reshape(b * h), table, bblk=32, bsz=b, hist=h)
